# Initial kernel scaffold; baseline (speedup 1.0000x reference)
import jax
import jax.numpy as jnp
from jax.experimental import pallas as pl


def kernel(x, edge_index, edge_attr, batch, Wl1, bl1, Wr1, br1, We1, att1, bg1, Wl2, bl2, Wr2, br2, We2, att2, bg2, Wl3, bl3, Wr3, br3, We3, att3, bg3, W4, b4, W5, b5):
    raise NotImplementedError("write your pallas kernel here")


# speedup vs baseline: 1.4162x; 1.4162x over previous
import functools

import jax
import jax.numpy as jnp
from jax import lax
from jax.experimental import pallas as pl
from jax.experimental.pallas import tpu as pltpu
from jax.experimental.pallas import tpu_sc as plsc

N = 10000
E = 640000
G = 64
H = 128

NC = 2
NS = 16
CHUNK = 64

NP = 10240
HALFN = NP // 2
RPT = HALFN // NS

E2 = E + N
CHUNKS_PER_TILE = 636
T_PER_TILE = CHUNKS_PER_TILE * CHUNK
E2P = NS * T_PER_TILE
GRP = NS * CHUNK


@functools.cache
def _mesh():
    return plsc.VectorSubcoreMesh(core_axis_name="c", subcore_axis_name="s",
                                  num_cores=NC)


def _zero_vmem_rows(ref, nrows, ncols16):
    zero = jnp.zeros((16,), jnp.float32)

    def body(r, carry):
        for cc in range(ncols16):
            ref[r, pl.ds(16 * cc, 16)] = zero
        return carry

    lax.fori_loop(0, nrows, body, 0)


def _cnt_for_core(cnts_v, c):
    cv = cnts_v[pl.ds(0, 16)]
    return jnp.where(c == 0, cv[0], cv[1])


@functools.cache
def _sc_segsum16_call():
    return functools.partial(
        pl.kernel,
        mesh=_mesh(),
        out_type=jax.ShapeDtypeStruct((NC, NS, HALFN * 16), jnp.float32),
        scratch_types=[
            pltpu.VMEM((CHUNK * 16,), jnp.float32),
            pltpu.VMEM((CHUNK,), jnp.int32),
            pltpu.VMEM((16,), jnp.int32),
            pltpu.VMEM((HALFN * 16,), jnp.float32),
        ],
    )(_sc_segsum16)


def _sc_segsum16(ea2p_hbm, dstg_hbm, cnts_hbm, out_hbm,
                 row_v, idx_v, cnts_v, acc_t):
    c = lax.axis_index("c")
    s = lax.axis_index("s")

    pltpu.sync_copy(cnts_hbm, cnts_v)
    cnt_c = _cnt_for_core(cnts_v, c)
    nch = (cnt_c + GRP - 1) // GRP

    zero16 = jnp.zeros((16,), jnp.float32)

    def zacc(b, carry):
        acc_t[pl.ds(16 * b, 16)] = zero16
        return carry

    lax.fori_loop(0, HALFN, zacc, 0)

    coff = c * HALFN

    def body(ci, carry):
        off = c * E2P + (ci * NS + s) * CHUNK
        pltpu.sync_copy(dstg_hbm.at[pl.ds(off, CHUNK)], idx_v)
        pltpu.sync_copy(ea2p_hbm.at[pl.ds(off * 16, CHUNK * 16)], row_v)

        def group(g, carry2):
            dvec = idx_v[pl.ds(g * 16, 16)] - coff
            for t in range(16):
                e = g * 16 + t
                d_s = dvec[t]
                sl = pl.ds(d_s * 16, 16)
                acc_t[sl] = acc_t[sl] + row_v[pl.ds(e * 16, 16)]
            return carry2

        lax.fori_loop(0, CHUNK // 16, group, 0)
        return carry

    lax.fori_loop(0, nch, body, 0)
    pltpu.sync_copy(acc_t, out_hbm.at[c, s])


@functools.cache
def _sc_edge_pass_call():
    return functools.partial(
        pl.kernel,
        mesh=_mesh(),
        out_type=[
            jax.ShapeDtypeStruct((NC, HALFN, H), jnp.float32),
            jax.ShapeDtypeStruct((NC * NS * HALFN,), jnp.float32),
        ],
        scratch_types=[
        pltpu.VMEM((CHUNK,), jnp.int32),
        pltpu.VMEM((CHUNK,), jnp.int32),
        pltpu.VMEM((CHUNK,), jnp.int32),
        pltpu.VMEM((CHUNK,), jnp.int32),
        pltpu.VMEM((CHUNK,), jnp.int32),
        pltpu.VMEM((CHUNK,), jnp.int32),
        pltpu.VMEM((CHUNK * 16,), jnp.float32),
        pltpu.VMEM((CHUNK * 16,), jnp.float32),
        pltpu.VMEM((CHUNK, H), jnp.float32),
        pltpu.VMEM((CHUNK, H), jnp.float32),
        pltpu.VMEM((CHUNK, H), jnp.float32),
        pltpu.VMEM((CHUNK, H), jnp.float32),
        pltpu.VMEM((CHUNK, H), jnp.float32),
        pltpu.VMEM((4, H), jnp.float32),
        pltpu.VMEM((H,), jnp.float32),
        pltpu.VMEM((16,), jnp.int32),
        pltpu.VMEM((HALFN,), jnp.float32),
        pltpu.VMEM_SHARED((HALFN, H), jnp.float32),
        pltpu.SemaphoreType.DMA,
        pltpu.SemaphoreType.DMA,
        pltpu.SemaphoreType.DMA,
        pltpu.SemaphoreType.DMA,
        ],
    )(_sc_edge_pass)


def _sc_edge_pass(xl_hbm, xr_hbm, src2_hbm, dstg_hbm, ea2_hbm, wet_hbm,
                  att_hbm, cnts_hbm,
                  out_hbm, den_hbm,
                  sidx0, didx0, sidx1, didx1, didxl0, didxl1, ea0, ea1,
                  xlb0, xrb0, xlb1, xrb1, wxl, wet_v, att_v, cnts_v,
                  den_v,
                  acc_sh, semxl0, semxr0, semxl1, semxr1):
    c = lax.axis_index("c")
    s = lax.axis_index("s")
    coff = c * HALFN

    pltpu.sync_copy(wet_hbm, wet_v)
    pltpu.sync_copy(att_hbm, att_v)
    pltpu.sync_copy(cnts_hbm, cnts_v)
    cnt_c = _cnt_for_core(cnts_v, c)
    nch = (cnt_c + GRP - 1) // GRP
    nchp = (nch + 1) // 2

    zero16 = jnp.zeros((16,), jnp.float32)
    _zero_vmem_rows(wxl, CHUNK, 8)

    def zden(b, carry):
        den_v[pl.ds(16 * b, 16)] = zero16
        return carry

    lax.fori_loop(0, HALFN // 16, zden, 0)

    for i in range(RPT // 64):
        pltpu.sync_copy(wxl.at[pl.ds(0, 64)],
                        acc_sh.at[pl.ds(s * RPT + i * 64, 64)])
    plsc.subcore_barrier()

    wets = [[wet_v[k, pl.ds(16 * j, 16)] for j in range(8)] for k in range(4)]
    atts = [att_v[pl.ds(16 * j, 16)] for j in range(8)]
    lanes = lax.iota(jnp.int32, 16)
    perms = [lanes ^ sh for sh in (8, 4, 2, 1)]

    def allsum(v):
        for p in perms:
            v = v + jnp.take_along_axis(v, p, axis=0)
        return v

    def exp16(x):
        t = x * (1.0 / 64.0)
        p = jnp.full((16,), 1.0 / 5040.0, jnp.float32)
        for cf in (1.0 / 720.0, 1.0 / 120.0, 1.0 / 24.0, 1.0 / 6.0,
                   0.5, 1.0, 1.0):
            p = p * t + cf
        for _ in range(6):
            p = p * p
        return p

    bufs = ((sidx0, didx0, didxl0, ea0, xlb0, xrb0, semxl0, semxr0),
            (sidx1, didx1, didxl1, ea1, xlb1, xrb1, semxl1, semxr1))

    def issue(ci, b):
        sidx, didx, didxl, eab, xlb, xrb, semxl, semxr = bufs[b]
        off = c * E2P + (ci * NS + s) * CHUNK
        pltpu.sync_copy(src2_hbm.at[pl.ds(off, CHUNK)], sidx)
        pltpu.sync_copy(dstg_hbm.at[pl.ds(off, CHUNK)], didx)
        pltpu.sync_copy(ea2_hbm.at[pl.ds(off * 16, CHUNK * 16)], eab)
        pltpu.async_copy(xl_hbm.at[sidx], xlb, semxl)
        pltpu.async_copy(xr_hbm.at[didx], xrb, semxr)

    def wait(b):
        sidx, didx, didxl, eab, xlb, xrb, semxl, semxr = bufs[b]
        pltpu.make_async_copy(xl_hbm.at[sidx], xlb, semxl).wait()
        pltpu.make_async_copy(xr_hbm.at[didx], xrb, semxr).wait()

    def compute_scatter(ci, b):
        sidx, didx, didxl, eab, xlb, xrb, semxl, semxr = bufs[b]
        lbase = (ci * NS + s) * CHUNK

        def group(g, carry):
            dvec = didx[pl.ds(g * 16, 16)]
            dloc = dvec - coff
            didxl[pl.ds(g * 16, 16)] = dloc
            for t in range(16):
                e = g * 16 + t
                row = eab[pl.ds(e * 16, 16)]
                a0, a1, a2, a3 = row[0], row[1], row[2], row[3]
                acc = None
                for j in range(8):
                    xlj = xlb[e, pl.ds(16 * j, 16)]
                    z = xlj + xrb[e, pl.ds(16 * j, 16)]
                    z = z + a0 * wets[0][j] + a1 * wets[1][j]
                    z = z + a2 * wets[2][j] + a3 * wets[3][j]
                    z = jnp.maximum(z, 0.2 * z)
                    term = atts[j] * z
                    acc = term if acc is None else acc + term
                wv = exp16(allsum(acc))
                ws = jnp.where(lbase + e < cnt_c, wv[0], 0.0)
                for j in range(8):
                    wxl[e, pl.ds(16 * j, 16)] = xlb[e, pl.ds(16 * j, 16)] * ws
                d_s = dloc[t]
                dbase = lax.bitwise_and(d_s, -16)
                drem = lax.bitwise_and(d_s, 15)
                dv = den_v[pl.ds(dbase, 16)]
                den_v[pl.ds(dbase, 16)] = jnp.where(lanes == drem, dv + ws, dv)
            return carry

        lax.fori_loop(0, CHUNK // 16, group, 0)
        pltpu.sync_copy(wxl, acc_sh.at[didxl], add=True)

    issue(0, 0)

    def body(ci2, carry):
        ci = ci2 * 2
        issue(ci + 1, 1)
        wait(0)
        compute_scatter(ci, 0)
        issue(lax.rem(ci + 2, 2 * nchp), 0)
        wait(1)
        compute_scatter(ci + 1, 1)
        return carry

    lax.fori_loop(0, nchp, body, 0)
    wait(0)
    pltpu.sync_copy(den_v, den_hbm.at[pl.ds((c * NS + s) * HALFN, HALFN)])
    plsc.subcore_barrier()

    r0 = s * RPT
    pltpu.sync_copy(acc_sh.at[pl.ds(r0, RPT)], out_hbm.at[c, pl.ds(r0, RPT)])


ROWB = 400
NBLK = N // ROWB


def _tc_first(x, la, wlT, bl, wrT, br):
    def body(x_ref, la_ref, wl_ref, bl_ref, wr_ref, br_ref,
             xl_ref, xr_ref, lao_ref):
        xb = x_ref[...].astype(jnp.bfloat16)
        xl_ref[...] = jnp.dot(xb, wl_ref[...].astype(jnp.bfloat16),
                              preferred_element_type=jnp.float32) + bl_ref[...]
        xr_ref[...] = jnp.dot(xb, wr_ref[...].astype(jnp.bfloat16),
                              preferred_element_type=jnp.float32) + br_ref[...]
        lab = jnp.sum(la_ref[...], axis=1)
        deg = lab[:, 4:5]
        lao_ref[...] = lab / jnp.maximum(deg, 1.0)

    return pl.pallas_call(
        body,
        grid=(NBLK,),
        in_specs=[
            pl.BlockSpec((ROWB, 11), lambda i: (i, 0)),
            pl.BlockSpec((ROWB, NS, 16), lambda i: (i, 0, 0)),
            pl.BlockSpec((11, H), lambda i: (0, 0)),
            pl.BlockSpec((1, H), lambda i: (0, 0)),
            pl.BlockSpec((11, H), lambda i: (0, 0)),
            pl.BlockSpec((1, H), lambda i: (0, 0)),
        ],
        out_specs=[
            pl.BlockSpec((ROWB, H), lambda i: (i, 0)),
            pl.BlockSpec((ROWB, H), lambda i: (i, 0)),
            pl.BlockSpec((ROWB, 16), lambda i: (i, 0)),
        ],
        out_shape=[
            jax.ShapeDtypeStruct((N, H), jnp.float32),
            jax.ShapeDtypeStruct((N, H), jnp.float32),
            jax.ShapeDtypeStruct((N, 16), jnp.float32),
        ],
    )(x, la, wlT, bl, wrT, br)


def _tc_combine_project(outp, denp, bg, wlT, bl, wrT, br):
    def body(p_ref, d_ref, bg_ref, wl_ref, bl_ref, wr_ref, br_ref,
             h_ref, xl_ref, xr_ref):
        o = p_ref[...]
        den = jnp.sum(d_ref[...], axis=1, keepdims=True)
        h = jnp.maximum(o / (den + 1e-16) + bg_ref[...], 0.0)
        h_ref[...] = h
        h16 = h.astype(jnp.bfloat16)
        xl_ref[...] = jnp.dot(h16, wl_ref[...].astype(jnp.bfloat16),
                              preferred_element_type=jnp.float32) + bl_ref[...]
        xr_ref[...] = jnp.dot(h16, wr_ref[...].astype(jnp.bfloat16),
                              preferred_element_type=jnp.float32) + br_ref[...]

    return pl.pallas_call(
        body,
        grid=(NBLK,),
        in_specs=[
            pl.BlockSpec((ROWB, H), lambda i: (i, 0)),
            pl.BlockSpec((ROWB, NS), lambda i: (i, 0)),
            pl.BlockSpec((1, H), lambda i: (0, 0)),
            pl.BlockSpec((H, H), lambda i: (0, 0)),
            pl.BlockSpec((1, H), lambda i: (0, 0)),
            pl.BlockSpec((H, H), lambda i: (0, 0)),
            pl.BlockSpec((1, H), lambda i: (0, 0)),
        ],
        out_specs=[
            pl.BlockSpec((ROWB, H), lambda i: (i, 0)),
            pl.BlockSpec((ROWB, H), lambda i: (i, 0)),
            pl.BlockSpec((ROWB, H), lambda i: (i, 0)),
        ],
        out_shape=[
            jax.ShapeDtypeStruct((N, H), jnp.float32),
            jax.ShapeDtypeStruct((N, H), jnp.float32),
            jax.ShapeDtypeStruct((N, H), jnp.float32),
        ],
    )(outp, denp, bg, wlT, bl, wrT, br)


def _tc_pool_head(h3, batch2d, w4T, b4, w5T, b5):
    def body(h_ref, b_ref, w4_ref, b4_ref, w5_ref, b5_ref, y_ref, g_acc):
        i = pl.program_id(0)

        @pl.when(i == 0)
        def _init():
            g_acc[...] = jnp.zeros((G, H), jnp.float32)

        h = h_ref[...]
        gids = lax.broadcasted_iota(jnp.int32, (ROWB, G), 1)
        onehot = (b_ref[...] == gids).astype(jnp.float32)
        g_acc[...] += lax.dot_general(onehot, h, (((0,), (0,)), ((), ())),
                                      preferred_element_type=jnp.float32,
                                      precision=lax.Precision.HIGHEST)

        @pl.when(i == NBLK - 1)
        def _head():
            g = g_acc[...]
            t = jnp.maximum(jnp.dot(g.astype(jnp.bfloat16),
                                    w4_ref[...].astype(jnp.bfloat16),
                                    preferred_element_type=jnp.float32)
                            + b4_ref[...], 0.0)
            y_ref[...] = jnp.dot(t.astype(jnp.bfloat16),
                                 w5_ref[...].astype(jnp.bfloat16),
                                 preferred_element_type=jnp.float32) + b5_ref[...]

    return pl.pallas_call(
        body,
        grid=(NBLK,),
        in_specs=[
            pl.BlockSpec((ROWB, H), lambda i: (i, 0)),
            pl.BlockSpec((ROWB, 1), lambda i: (i, 0)),
            pl.BlockSpec((H, G), lambda i: (0, 0)),
            pl.BlockSpec((1, G), lambda i: (0, 0)),
            pl.BlockSpec((G, 1), lambda i: (0, 0)),
            pl.BlockSpec((1, 1), lambda i: (0, 0)),
        ],
        out_specs=pl.BlockSpec((G, 1), lambda i: (0, 0)),
        out_shape=jax.ShapeDtypeStruct((G, 1), jnp.float32),
        scratch_shapes=[pltpu.VMEM((G, H), jnp.float32)],
    )(h3, batch2d, w4T, b4, w5T, b5)


def kernel(x, edge_index, edge_attr, batch,
           Wl1, bl1, Wr1, br1, We1, att1, bg1,
           Wl2, bl2, Wr2, br2, We2, att2, bg2,
           Wl3, bl3, Wr3, br3, We3, att3, bg3,
           W4, b4, W5, b5):
    f32 = jnp.float32
    i32 = jnp.int32
    src = edge_index[0].astype(i32)
    dst = edge_index[1].astype(i32)
    ea = edge_attr.astype(f32)

    loop = jnp.arange(N, dtype=i32)
    src2 = jnp.concatenate([src, loop])
    dst2 = jnp.concatenate([dst, loop])
    key = (dst2 >= HALFN).astype(i32)
    incl = jnp.cumsum(key)
    cnt1 = incl[-1]
    cnt0 = E2 - cnt1
    pos = jnp.arange(E2, dtype=i32)
    idx_t = jnp.where(key == 0, pos - incl, E2P + incl - 1)

    src2p = jnp.zeros((2 * E2P,), i32).at[idx_t].set(src2)
    posoff = jnp.where(jnp.arange(2 * E2P, dtype=i32) >= E2P, HALFN, 0)
    dstg = posoff.at[idx_t].set(dst2)
    def bfround(a):
        return a.astype(jnp.bfloat16).astype(f32)

    ea16 = jnp.concatenate(
        [ea, jnp.ones((E, 1), f32), jnp.zeros((E, 11), f32)], axis=1)
    ea2p_seg = jnp.zeros((2 * E2P, 16), f32).at[idx_t[:E]].set(ea16).reshape(-1)
    ea16r = jnp.concatenate([bfround(ea), jnp.zeros((E, 12), f32)], axis=1)
    ea2p = jnp.zeros((2 * E2P, 16), f32).at[idx_t[:E]].set(ea16r)
    cnts16 = jnp.zeros((16,), i32).at[0].set(cnt0).at[1].set(cnt1)

    laparts = _sc_segsum16_call()(ea2p_seg, dstg, cnts16)
    la = (laparts.reshape(NC, NS, HALFN, 16)
          .transpose(0, 2, 1, 3).reshape(NP, NS, 16)[:N])

    xl, xr, la16 = _tc_first(
        x.astype(f32), la,
        Wl1.T.astype(f32), bl1.reshape(1, H).astype(f32),
        Wr1.T.astype(f32), br1.reshape(1, H).astype(f32))
    loop16 = jnp.concatenate(
        [bfround(la16[:, :4]), jnp.zeros((N, 12), f32)], axis=1)
    ea2p = ea2p.at[idx_t[E:]].set(loop16).reshape(-1)

    zH = jnp.zeros((H, H), f32)
    z1 = jnp.zeros((1, H), f32)
    wetT_all = bfround(jnp.stack([We1.T, We2.T, We3.T]).astype(f32))
    att_all = jnp.stack([att1, att2, att3]).astype(f32)
    bg_all = jnp.stack([bg1.reshape(1, H), bg2.reshape(1, H),
                        bg3.reshape(1, H)]).astype(f32)
    wlT_n = jnp.stack([Wl2.T.astype(f32), Wl3.T.astype(f32), zH])
    bl_n = jnp.stack([bl2.reshape(1, H).astype(f32),
                      bl3.reshape(1, H).astype(f32), z1])
    wrT_n = jnp.stack([Wr2.T.astype(f32), Wr3.T.astype(f32), zH])
    br_n = jnp.stack([br2.reshape(1, H).astype(f32),
                      br3.reshape(1, H).astype(f32), z1])

    def lbody(l, carry):
        xl_c, xr_c, _ = carry
        parts, dens = _sc_edge_pass_call()(xl_c, xr_c, src2p, dstg, ea2p,
                                           wetT_all[l], att_all[l], cnts16)
        outp = parts.reshape(NP, H)[:N]
        denp = dens.reshape(NC, NS, HALFN).transpose(0, 2, 1).reshape(NP, NS)[:N]
        h, xl_n, xr_n = _tc_combine_project(
            outp, denp, bg_all[l],
            wlT_n[l], bl_n[l], wrT_n[l], br_n[l])
        return (xl_n, xr_n, h)

    _, _, h3 = lax.fori_loop(0, 3, lbody,
                             (xl, xr, jnp.zeros((N, H), f32)))

    y = _tc_pool_head(
        h3, batch.astype(i32).reshape(N, 1),
        W4.T.astype(f32), b4.reshape(1, G).astype(f32),
        W5.T.astype(f32), b5.reshape(1, 1).astype(f32))
    return y
